# Initial kernel scaffold; baseline (speedup 1.0000x reference)
import jax
import jax.numpy as jnp
from jax.experimental import pallas as pl


def kernel(x, pos_table):
    raise NotImplementedError("write your pallas kernel here")


# speedup vs baseline: 2.4113x; 2.4113x over previous
import jax
import jax.numpy as jnp
from jax.experimental import pallas as pl

_ROWS = 8192
_COLS = 1024
_BLOCK_ROWS = 1024


def _add_block(x_ref, p_ref, o_ref):
    o_ref[...] = x_ref[...] + p_ref[...]


def kernel(x, pos_table):
    n = x.shape[0]
    spec = pl.BlockSpec((_BLOCK_ROWS, _COLS), lambda i: (i, 0))
    return pl.pallas_call(
        _add_block,
        grid=(n // _BLOCK_ROWS,),
        in_specs=[spec, spec],
        out_specs=spec,
        out_shape=jax.ShapeDtypeStruct((n, _COLS), x.dtype),
    )(x, pos_table[:n])
